# baseline (speedup 1.0000x reference)
import functools

import jax
import jax.numpy as jnp
from jax import lax
from jax.experimental import pallas as pl
from jax.experimental.pallas import tpu as pltpu
from jax.experimental.pallas import tpu_sc as plsc

BATCH = 16384
DIM = 16
NC = 2
NS = 16
NW = NC * NS
B_PER_W = BATCH // NW
CHUNK = 128
NCHUNK = B_PER_W // CHUNK

_mesh = plsc.VectorSubcoreMesh(
    core_axis_name="c", subcore_axis_name="s", num_cores=NC, num_subcores=NS)


@functools.partial(
    pl.kernel,
    mesh=_mesh,
    out_type=(
        jax.ShapeDtypeStruct((BATCH, DIM), jnp.float32),
        jax.ShapeDtypeStruct((BATCH, DIM), jnp.float32),
    ),
    scratch_types=(
        pltpu.VMEM((NCHUNK, CHUNK), jnp.int32),
        pltpu.VMEM((NCHUNK, CHUNK), jnp.int32),
        pltpu.VMEM((NCHUNK, CHUNK), jnp.int32),
        pltpu.VMEM((B_PER_W, DIM), jnp.float32),
        pltpu.VMEM((B_PER_W, DIM), jnp.float32),
        pltpu.VMEM((B_PER_W, DIM), jnp.float32),
        pltpu.SemaphoreType.DMA,
    ),
    compiler_params=pltpu.CompilerParams(use_tc_tiling_on_sc=False),
)
def _bdl_fwd(user_hbm, item_hbm, neg_hbm, uw_hbm, iw_hbm,
             out_ui, out_uj, idx_u, idx_i, idx_j, ru, ri, rj, sem):
    wid = lax.axis_index("s") * NC + lax.axis_index("c")
    base = wid * B_PER_W

    pltpu.sync_copy(user_hbm.at[wid], idx_u)
    pltpu.sync_copy(item_hbm.at[wid], idx_i)
    pltpu.sync_copy(neg_hbm.at[wid], idx_j)

    copies = []
    for c in range(NCHUNK):
        sl = pl.ds(c * CHUNK, CHUNK)
        copies.append(pltpu.async_copy(uw_hbm.at[idx_u.at[c]], ru.at[sl], sem))
        copies.append(pltpu.async_copy(iw_hbm.at[idx_i.at[c]], ri.at[sl], sem))
        copies.append(pltpu.async_copy(iw_hbm.at[idx_j.at[c]], rj.at[sl], sem))
    for cp in copies:
        cp.wait()

    def body(r, _):
        u = ru[r, :]
        ri[r, :] = u * ri[r, :]
        rj[r, :] = u * rj[r, :]
        return 0

    lax.fori_loop(0, B_PER_W, body, 0)

    pltpu.sync_copy(ri, out_ui.at[pl.ds(base, B_PER_W)])
    pltpu.sync_copy(rj, out_uj.at[pl.ds(base, B_PER_W)])


def kernel(user, item, neg_item, user_emb_w, item_emb_w):
    u3 = user.astype(jnp.int32).reshape(NW, NCHUNK, CHUNK)
    i3 = item.astype(jnp.int32).reshape(NW, NCHUNK, CHUNK)
    j3 = neg_item.astype(jnp.int32).reshape(NW, NCHUNK, CHUNK)
    return _bdl_fwd(u3, i3, j3, user_emb_w, item_emb_w)
